# baseline (speedup 1.0000x reference)
import functools

import jax
import jax.numpy as jnp
from jax import lax
from jax.experimental import pallas as pl
from jax.experimental.pallas import tpu as pltpu
from jax.experimental.pallas import tpu_sc as plsc

_N = 10000
_NP = 10112
_D = 256
_B = 64
_E = 160000
_ETOT = _E + _N
_NSUB = 16
_NCORE = 2
_BLK = 48
_NBLK = 224
_NBLK2 = 112
_EP = _NSUB * _NBLK * _BLK
_RPS = _NP // _NSUB
_ACCW = 136
_MM_BM = 1264

_sc_mesh = plsc.VectorSubcoreMesh(
    core_axis_name="c", subcore_axis_name="s",
    num_cores=_NCORE, num_subcores=_NSUB)

_CHUNKS = tuple((i * 48, 48) for i in range(13)) + ((624, 8),)


def _mm1_body(x_ref, w_ref, o_ref):
    y = jnp.dot(x_ref[...], w_ref[...], preferred_element_type=jnp.float32)
    o_ref[0] = y[:, 0:128]
    o_ref[1] = y[:, 128:256]
    o_ref[2] = y[:, 256:384]
    o_ref[3] = y[:, 384:512]


def _mm1(x_pad, wcat):
    return pl.pallas_call(
        _mm1_body,
        grid=(_NP // _MM_BM,),
        in_specs=[
            pl.BlockSpec((_MM_BM, _D), lambda i: (i, 0)),
            pl.BlockSpec((_D, 512), lambda i: (0, 0)),
        ],
        out_specs=pl.BlockSpec((4, _MM_BM, 128), lambda i: (0, i, 0)),
        out_shape=jax.ShapeDtypeStruct((4, _NP, 128), jnp.float32),
    )(x_pad, wcat)


def _mm2_body(h_ref, w_ref, o_ref):
    y0 = jnp.dot(h_ref[0], w_ref[0], preferred_element_type=jnp.float32)
    y1 = jnp.dot(h_ref[1], w_ref[1], preferred_element_type=jnp.float32)
    o_ref[...] = y0 + y1


def _mm2(h1, w2cat):
    return pl.pallas_call(
        _mm2_body,
        grid=(_NP // _MM_BM,),
        in_specs=[
            pl.BlockSpec((2, _MM_BM, 128), lambda i: (0, i, 0)),
            pl.BlockSpec((2, 128, 32), lambda i: (0, 0, 0)),
        ],
        out_specs=pl.BlockSpec((_MM_BM, 32), lambda i: (i, 0)),
        out_shape=jax.ShapeDtypeStruct((_NP, 32), jnp.float32),
    )(h1, w2cat)


@functools.partial(
    pl.kernel,
    out_type=jax.ShapeDtypeStruct((_NCORE * _NP, 128), jnp.float32),
    mesh=_sc_mesh,
    compiler_params=pltpu.CompilerParams(
        needs_layout_passes=False, use_tc_tiling_on_sc=False),
    scratch_types=[
        pltpu.VMEM_SHARED((_NP, _ACCW), jnp.float32),
        pltpu.VMEM((2 * _BLK, 128), jnp.float32),
        pltpu.VMEM((2 * _BLK, 128), jnp.float32),
        pltpu.VMEM((_BLK, _ACCW), jnp.float32),
        pltpu.VMEM((_BLK, _ACCW), jnp.float32),
        pltpu.VMEM((2 * _BLK,), jnp.int32),
        pltpu.VMEM((2 * _BLK,), jnp.int32),
        pltpu.VMEM((_BLK,), jnp.int32),
        pltpu.VMEM((2 * _BLK,), jnp.int32),
        pltpu.VMEM((2 * _BLK,), jnp.int32),
        pltpu.VMEM((_BLK,), jnp.int32),
        pltpu.VMEM((128,), jnp.float32),
        pltpu.VMEM((128,), jnp.float32),
        pltpu.SemaphoreType.DMA,
        pltpu.SemaphoreType.DMA,
        pltpu.SemaphoreType.DMA,
        pltpu.SemaphoreType.DMA,
        pltpu.SemaphoreType.DMA,
        pltpu.SemaphoreType.DMA,
    ],
)
def _l1(xlxr_hbm, cidx_hbm, att_hbm, b1_hbm, h1_hbm,
        acc, gbA, gbB, wA, wB,
        ciA, caA, dwA, ciB, caB, dwB,
        attb, biasb,
        semA, semB, semWA, semWB, semIA, semIB):
    c = lax.axis_index("c")
    s = lax.axis_index("s")
    iota = lax.iota(jnp.int32, 16)
    zv = jnp.zeros((16,), jnp.float32)
    coff = c * _NP
    r0 = s * _RPS

    pltpu.sync_copy(att_hbm.at[c], attb)
    pltpu.sync_copy(b1_hbm.at[c], biasb)
    attvs = [attb[pl.ds(16 * j, 16)] for j in range(8)]
    biasvs = [biasb[pl.ds(16 * j, 16)] for j in range(8)]

    @plsc.parallel_loop(0, _BLK, unroll=2)
    def _zw(r):
        for j in range(8):
            wA[r, pl.ds(16 * j, 16)] = zv
        wA[r, pl.ds(120, 16)] = zv
    for (o, n) in _CHUNKS:
        pltpu.sync_copy(wA.at[pl.ds(0, n)], acc.at[pl.ds(r0 + o, n)])
    plsc.subcore_barrier()

    def _idx_load(b, cix, semi):
        pltpu.async_copy(cidx_hbm.at[s, b], cix, semi)

    def _idx_wait(b, cix, semi):
        pltpu.make_async_copy(cidx_hbm.at[s, b], cix, semi).wait()

    def _adj(cix, cax):
        for j in range(2 * _BLK // 16):
            cax[pl.ds(16 * j, 16)] = cix[pl.ds(16 * j, 16)] + coff

    def _cp_dw(cix, dwx):
        for j in range(_BLK // 16):
            dwx[pl.ds(16 * j, 16)] = cix[pl.ds(_BLK + 16 * j, 16)] - 2 * _NP

    def _start_g(cax, gbx, s1):
        pltpu.async_copy(xlxr_hbm.at[cax], gbx, s1)

    def _wait_g(cax, gbx, s1):
        pltpu.make_async_copy(xlxr_hbm.at[cax], gbx, s1).wait()

    def _compute(gbx, wx):
        @plsc.parallel_loop(0, _BLK, unroll=4)
        def _edge(e):
            xls = []
            acc0 = zv
            acc1 = zv
            for j in range(8):
                xlv = gbx[e, pl.ds(16 * j, 16)]
                xrv = gbx[_BLK + e, pl.ds(16 * j, 16)]
                xls.append(xlv)
                v = xlv + xrv
                f = jnp.where(v >= 0.0, v, 0.2 * v)
                p = f * attvs[j]
                if j < 4:
                    acc0 = acc0 + p
                else:
                    acc1 = acc1 + p
            l0 = jnp.sum(acc0)
            l1 = jnp.sum(acc1)
            e0 = jnp.exp(jnp.full((16,), l0, jnp.float32))
            e1 = jnp.exp(jnp.full((16,), l1, jnp.float32))
            dv = jnp.where(iota == 0, e0, jnp.where(iota == 1, e1, zv))
            wx[e, pl.ds(0, 16)] = dv
            for j in range(8):
                wx[e, pl.ds(8 + 16 * j, 16)] = xls[j] * (e0 if j < 4 else e1)

    def _scat(wx, dwx, semw):
        pltpu.async_copy(wx, acc.at[dwx], semw, add=True)

    def _wait_s(wx, dwx, semw):
        pltpu.make_async_copy(wx, acc.at[dwx], semw).wait()

    pltpu.sync_copy(cidx_hbm.at[s, 0], ciA)
    _adj(ciA, caA)
    pltpu.sync_copy(cidx_hbm.at[s, 1], ciB)
    _adj(ciB, caB)
    _start_g(caA, gbA, semA)

    nstep = _NBLK // 2

    def _step(g, carry):
        pA = 2 * g
        pB = 2 * g + 1
        _start_g(caB, gbB, semB)
        _wait_g(caA, gbA, semA)

        @pl.when(g > 0)
        def _wsa():
            _wait_s(wA, dwA, semWA)
        _cp_dw(ciA, dwA)

        @pl.when(g < nstep - 1)
        def _ila():
            _idx_load(pA + 2, ciA, semIA)
        _compute(gbA, wA)
        _scat(wA, dwA, semWA)

        @pl.when(g < nstep - 1)
        def _iwa():
            _idx_wait(pA + 2, ciA, semIA)
            _adj(ciA, caA)
            _start_g(caA, gbA, semA)

        _wait_g(caB, gbB, semB)

        @pl.when(g > 0)
        def _wsb():
            _wait_s(wB, dwB, semWB)
        _cp_dw(ciB, dwB)

        @pl.when(g < nstep - 1)
        def _ilb():
            _idx_load(pB + 2, ciB, semIB)
        _compute(gbB, wB)
        _scat(wB, dwB, semWB)

        @pl.when(g < nstep - 1)
        def _iwb():
            _idx_wait(pB + 2, ciB, semIB)
            _adj(ciB, caB)
        return carry
    lax.fori_loop(0, nstep, _step, 0)
    _wait_s(wA, dwA, semWA)
    _wait_s(wB, dwB, semWB)
    plsc.subcore_barrier()

    for (o, n) in _CHUNKS:
        pltpu.sync_copy(acc.at[pl.ds(r0 + o, n)], wA.at[pl.ds(0, n)])

        @plsc.parallel_loop(0, n, unroll=2)
        def _row(r):
            denv = wA[r, pl.ds(0, 16)]
            iv = 1.0 / jnp.maximum(denv, 1e-16)
            i0 = iv[0]
            i1 = iv[1]
            for j in range(8):
                sc = i0 if j < 4 else i1
                v = wA[r, pl.ds(8 + 16 * j, 16)] * sc + biasvs[j]
                gbA[r, pl.ds(16 * j, 16)] = jnp.maximum(v, 0.0)
        pltpu.sync_copy(gbA.at[pl.ds(0, n)],
                        h1_hbm.at[pl.ds(coff + r0 + o, n)])


@functools.partial(
    pl.kernel,
    out_type=jax.ShapeDtypeStruct((_NCORE * _NP, 16), jnp.float32),
    mesh=_sc_mesh,
    compiler_params=pltpu.CompilerParams(
        needs_layout_passes=False, use_tc_tiling_on_sc=False),
    scratch_types=[
        pltpu.VMEM_SHARED((_NP, 16), jnp.float32),
        pltpu.VMEM((2 * _BLK, 32), jnp.float32),
        pltpu.VMEM((2 * _BLK, 32), jnp.float32),
        pltpu.VMEM((_BLK, 16), jnp.float32),
        pltpu.VMEM((_BLK, 16), jnp.float32),
        pltpu.VMEM((2 * _BLK,), jnp.int32),
        pltpu.VMEM((_BLK,), jnp.int32),
        pltpu.VMEM((2 * _BLK,), jnp.int32),
        pltpu.VMEM((_BLK,), jnp.int32),
        pltpu.VMEM((16,), jnp.float32),
        pltpu.SemaphoreType.DMA,
        pltpu.SemaphoreType.DMA,
        pltpu.SemaphoreType.DMA,
        pltpu.SemaphoreType.DMA,
        pltpu.SemaphoreType.DMA,
        pltpu.SemaphoreType.DMA,
    ],
)
def _l2(xlr2_hbm, cidx_hbm, att_hbm, out_hbm,
        acc2, gbA, gbB, w2A, w2B,
        ciA, dwA, ciB, dwB, attb,
        semA, semB, semWA, semWB, semIA, semIB):
    c = lax.axis_index("c")
    s = lax.axis_index("s")
    iota = lax.iota(jnp.int32, 16)
    zv = jnp.zeros((16,), jnp.float32)
    r0 = s * _RPS
    wid = s * _NCORE + c

    pltpu.sync_copy(att_hbm, attb)
    attv = attb[pl.ds(0, 16)]

    @plsc.parallel_loop(0, _BLK, unroll=4)
    def _zw(r):
        w2A[r, pl.ds(0, 16)] = zv
    for (o, n) in _CHUNKS:
        pltpu.sync_copy(w2A.at[pl.ds(0, n)], acc2.at[pl.ds(r0 + o, n)])
    plsc.subcore_barrier()

    def _idx_load(b, cix, semi):
        pltpu.async_copy(cidx_hbm.at[wid, b], cix, semi)

    def _idx_wait(b, cix, semi):
        pltpu.make_async_copy(cidx_hbm.at[wid, b], cix, semi).wait()

    def _cp_dw(cix, dwx):
        for j in range(_BLK // 16):
            dwx[pl.ds(16 * j, 16)] = cix[pl.ds(_BLK + 16 * j, 16)]

    def _start_g(cix, gbx, s1):
        pltpu.async_copy(xlr2_hbm.at[cix], gbx, s1)

    def _wait_g(cix, gbx, s1):
        pltpu.make_async_copy(xlr2_hbm.at[cix], gbx, s1).wait()

    def _compute(gbx, w2x):
        @plsc.parallel_loop(0, _BLK, unroll=2)
        def _edge(e):
            a = gbx[e, pl.ds(0, 16)]
            b8 = gbx[_BLK + e, pl.ds(8, 16)]
            v = a + b8
            f = jnp.where(v >= 0.0, v, 0.2 * v)
            exv = jnp.exp(f * attv)
            row = jnp.where(iota < 4, exv * a, jnp.where(iota < 8, exv, zv))
            w2x[e, pl.ds(0, 16)] = row

    def _scat(w2x, dwx, semw):
        pltpu.async_copy(w2x, acc2.at[dwx], semw, add=True)

    def _wait_s(w2x, dwx, semw):
        pltpu.make_async_copy(w2x, acc2.at[dwx], semw).wait()

    pltpu.sync_copy(cidx_hbm.at[wid, 0], ciA)
    pltpu.sync_copy(cidx_hbm.at[wid, 1], ciB)
    _start_g(ciA, gbA, semA)

    nstep = _NBLK2 // 2

    def _step(g, carry):
        pA = 2 * g
        pB = 2 * g + 1
        _start_g(ciB, gbB, semB)
        _wait_g(ciA, gbA, semA)

        @pl.when(g > 0)
        def _wsa():
            _wait_s(w2A, dwA, semWA)
        _cp_dw(ciA, dwA)

        @pl.when(g < nstep - 1)
        def _ila():
            _idx_load(pA + 2, ciA, semIA)
        _compute(gbA, w2A)
        _scat(w2A, dwA, semWA)

        @pl.when(g < nstep - 1)
        def _iwa():
            _idx_wait(pA + 2, ciA, semIA)
            _start_g(ciA, gbA, semA)

        _wait_g(ciB, gbB, semB)

        @pl.when(g > 0)
        def _wsb():
            _wait_s(w2B, dwB, semWB)
        _cp_dw(ciB, dwB)

        @pl.when(g < nstep - 1)
        def _ilb():
            _idx_load(pB + 2, ciB, semIB)
        _compute(gbB, w2B)
        _scat(w2B, dwB, semWB)

        @pl.when(g < nstep - 1)
        def _iwb():
            _idx_wait(pB + 2, ciB, semIB)
        return carry
    lax.fori_loop(0, nstep, _step, 0)
    _wait_s(w2A, dwA, semWA)
    _wait_s(w2B, dwB, semWB)
    plsc.subcore_barrier()

    for (o, n) in _CHUNKS:
        pltpu.sync_copy(acc2.at[pl.ds(r0 + o, n)], w2A.at[pl.ds(0, n)])
        pltpu.sync_copy(w2A.at[pl.ds(0, n)],
                        out_hbm.at[pl.ds(c * _NP + r0 + o, n)])


def _pool_body(p_ref, b_ref, b2_ref, o_ref):
    p = p_ref[0] + p_ref[1]
    num = p[:, 0:4]
    den = p[:, 4:8]
    vals = num / jnp.maximum(den, 1e-16)
    v = jnp.sum(vals, axis=1) * 0.25 + b2_ref[0, 0]
    bins = lax.broadcasted_iota(jnp.int32, (_B, _NP), 0)
    m = (b_ref[...] == bins).astype(jnp.float32)
    sums = jnp.sum(m * v[None, :], axis=1, keepdims=True)
    counts = jnp.sum(m, axis=1, keepdims=True)
    o_ref[...] = sums / jnp.maximum(counts, 1.0)


def _pool(parts, batch_p, b2v):
    return pl.pallas_call(
        _pool_body,
        grid=(1,),
        in_specs=[
            pl.BlockSpec((2, _NP, 16), lambda i: (0, 0, 0)),
            pl.BlockSpec((1, _NP), lambda i: (0, 0)),
            pl.BlockSpec((1, 1), lambda i: (0, 0)),
        ],
        out_specs=pl.BlockSpec((_B, 1), lambda i: (0, 0)),
        out_shape=jax.ShapeDtypeStruct((_B, 1), jnp.float32),
    )(parts, batch_p, b2v)


def kernel(x, edge_index, batch, Wl1, Wr1, att1, b1, Wl2, Wr2, att2, b2):
    ei = edge_index.astype(jnp.int32)
    loop = jnp.arange(_N, dtype=jnp.int32)
    src = jnp.concatenate([ei[0], loop])
    dst = jnp.concatenate([ei[1], loop])
    src_p = jnp.full((_EP,), _N, jnp.int32).at[:_ETOT].set(src)
    dst_p = jnp.full((_EP,), _N, jnp.int32).at[:_ETOT].set(dst)
    x_pad = jnp.zeros((_NP, _D), jnp.float32).at[:_N, :].set(x)

    cidx1 = jnp.concatenate(
        [src_p.reshape(_NSUB, _NBLK, _BLK),
         dst_p.reshape(_NSUB, _NBLK, _BLK) + 2 * _NP], axis=2)
    cidx2 = jnp.concatenate(
        [src_p.reshape(_NSUB * _NCORE, _NBLK2, _BLK),
         dst_p.reshape(_NSUB * _NCORE, _NBLK2, _BLK)], axis=2)

    wcat = jnp.concatenate([Wl1, Wr1], axis=1)
    xlxr = _mm1(x_pad, wcat)

    attc = att1.reshape(2, 128)
    b1c = b1.reshape(2, 128)
    h1cat = _l1(xlxr.reshape(4 * _NP, 128), cidx1, attc, b1c)
    h1 = h1cat.reshape(_NCORE, _NP, 128)

    w2cat = jnp.concatenate(
        [Wl2, Wl2, Wr2, Wr2, jnp.zeros((_D, 16), jnp.float32)], axis=1
    ).reshape(2, 128, 32)
    xlr2 = _mm2(h1, w2cat)

    att2p = (jnp.zeros((16,), jnp.float32)
             .at[:4].set(att2[:, 0]).at[4:8].set(att2[:, 0]))
    out2 = _l2(xlr2, cidx2, att2p)

    batch_p = jnp.full((1, _NP), _B, jnp.int32).at[0, :_N].set(
        batch.astype(jnp.int32))
    return _pool(out2.reshape(_NCORE, _NP, 16), batch_p, b2.reshape(1, 1))


# speedup vs baseline: 1.0471x; 1.0471x over previous
import functools

import jax
import jax.numpy as jnp
from jax import lax
from jax.experimental import pallas as pl
from jax.experimental.pallas import tpu as pltpu
from jax.experimental.pallas import tpu_sc as plsc

_N = 10000
_NP = 10112
_D = 256
_B = 64
_E = 160000
_ETOT = _E + _N
_NSUB = 16
_NCORE = 2
_BLK = 48
_NBLK = 224
_NBLK2 = 112
_EP = _NSUB * _NBLK * _BLK
_RPS = _NP // _NSUB
_ACCW = 136
_MM_BM = 1264

_sc_mesh = plsc.VectorSubcoreMesh(
    core_axis_name="c", subcore_axis_name="s",
    num_cores=_NCORE, num_subcores=_NSUB)

_CHUNKS = tuple((i * 48, 48) for i in range(13)) + ((624, 8),)


def _mm1_body(x_ref, w_ref, o_ref):
    y = jnp.dot(x_ref[...], w_ref[...], preferred_element_type=jnp.float32)
    o_ref[0] = y[:, 0:128]
    o_ref[1] = y[:, 128:256]
    o_ref[2] = y[:, 256:384]
    o_ref[3] = y[:, 384:512]


def _mm1(x_pad, wcat):
    return pl.pallas_call(
        _mm1_body,
        grid=(_NP // _MM_BM,),
        in_specs=[
            pl.BlockSpec((_MM_BM, _D), lambda i: (i, 0)),
            pl.BlockSpec((_D, 512), lambda i: (0, 0)),
        ],
        out_specs=pl.BlockSpec((4, _MM_BM, 128), lambda i: (0, i, 0)),
        out_shape=jax.ShapeDtypeStruct((4, _NP, 128), jnp.float32),
    )(x_pad, wcat)


def _mm2_body(h_ref, w_ref, o_ref):
    y0 = jnp.dot(h_ref[0], w_ref[0], preferred_element_type=jnp.float32)
    y1 = jnp.dot(h_ref[1], w_ref[1], preferred_element_type=jnp.float32)
    o_ref[...] = y0 + y1


def _mm2(h1, w2cat):
    return pl.pallas_call(
        _mm2_body,
        grid=(_NP // _MM_BM,),
        in_specs=[
            pl.BlockSpec((2, _MM_BM, 128), lambda i: (0, i, 0)),
            pl.BlockSpec((2, 128, 32), lambda i: (0, 0, 0)),
        ],
        out_specs=pl.BlockSpec((_MM_BM, 32), lambda i: (i, 0)),
        out_shape=jax.ShapeDtypeStruct((_NP, 32), jnp.float32),
    )(h1, w2cat)


@functools.partial(
    pl.kernel,
    out_type=jax.ShapeDtypeStruct((_NCORE * _NP, 128), jnp.float32),
    mesh=_sc_mesh,
    compiler_params=pltpu.CompilerParams(
        needs_layout_passes=False, use_tc_tiling_on_sc=False),
    scratch_types=[
        pltpu.VMEM_SHARED((_NP, _ACCW), jnp.float32),
        pltpu.VMEM((2 * _BLK, 128), jnp.float32),
        pltpu.VMEM((2 * _BLK, 128), jnp.float32),
        pltpu.VMEM((_BLK, _ACCW), jnp.float32),
        pltpu.VMEM((_BLK, _ACCW), jnp.float32),
        pltpu.VMEM((2 * _BLK,), jnp.int32),
        pltpu.VMEM((2 * _BLK,), jnp.int32),
        pltpu.VMEM((_BLK,), jnp.int32),
        pltpu.VMEM((2 * _BLK,), jnp.int32),
        pltpu.VMEM((2 * _BLK,), jnp.int32),
        pltpu.VMEM((_BLK,), jnp.int32),
        pltpu.VMEM((128,), jnp.float32),
        pltpu.VMEM((128,), jnp.float32),
        pltpu.SemaphoreType.DMA,
        pltpu.SemaphoreType.DMA,
        pltpu.SemaphoreType.DMA,
        pltpu.SemaphoreType.DMA,
        pltpu.SemaphoreType.DMA,
        pltpu.SemaphoreType.DMA,
    ],
)
def _l1(xlxr_hbm, cidx_hbm, att_hbm, b1_hbm, h1_hbm,
        acc, gbA, gbB, wA, wB,
        ciA, caA, dwA, ciB, caB, dwB,
        attb, biasb,
        semA, semB, semWA, semWB, semIA, semIB):
    c = lax.axis_index("c")
    s = lax.axis_index("s")
    iota = lax.iota(jnp.int32, 16)
    zv = jnp.zeros((16,), jnp.float32)
    coff = c * _NP
    r0 = s * _RPS

    pltpu.sync_copy(att_hbm.at[c], attb)
    pltpu.sync_copy(b1_hbm.at[c], biasb)
    attvs = [attb[pl.ds(16 * j, 16)] for j in range(8)]
    biasvs = [biasb[pl.ds(16 * j, 16)] for j in range(8)]

    @plsc.parallel_loop(0, _BLK, unroll=2)
    def _zw(r):
        for j in range(8):
            wA[r, pl.ds(16 * j, 16)] = zv
        wA[r, pl.ds(120, 16)] = zv
    for (o, n) in _CHUNKS:
        pltpu.sync_copy(wA.at[pl.ds(0, n)], acc.at[pl.ds(r0 + o, n)])
    plsc.subcore_barrier()

    def _idx_load(b, cix, semi):
        pltpu.async_copy(cidx_hbm.at[s, b], cix, semi)

    def _idx_wait(b, cix, semi):
        pltpu.make_async_copy(cidx_hbm.at[s, b], cix, semi).wait()

    def _adj(cix, cax):
        for j in range(2 * _BLK // 16):
            cax[pl.ds(16 * j, 16)] = cix[pl.ds(16 * j, 16)] + coff

    def _cp_dw(cix, dwx):
        for j in range(_BLK // 16):
            dwx[pl.ds(16 * j, 16)] = cix[pl.ds(_BLK + 16 * j, 16)] - 2 * _NP

    def _start_g(cax, gbx, s1):
        pltpu.async_copy(xlxr_hbm.at[cax], gbx, s1)

    def _wait_g(cax, gbx, s1):
        pltpu.make_async_copy(xlxr_hbm.at[cax], gbx, s1).wait()

    def _compute(gbx, wx):
        @plsc.parallel_loop(0, _BLK, unroll=2)
        def _edge(e):
            xls = []
            acc0 = zv
            acc1 = zv
            for j in range(8):
                xlv = gbx[e, pl.ds(16 * j, 16)]
                xrv = gbx[_BLK + e, pl.ds(16 * j, 16)]
                xls.append(xlv)
                v = xlv + xrv
                f = jnp.where(v >= 0.0, v, 0.2 * v)
                p = f * attvs[j]
                if j < 4:
                    acc0 = acc0 + p
                else:
                    acc1 = acc1 + p
            l0 = jnp.sum(acc0)
            l1 = jnp.sum(acc1)
            e0 = jnp.exp(jnp.full((16,), l0, jnp.float32))
            e1 = jnp.exp(jnp.full((16,), l1, jnp.float32))
            dv = jnp.where(iota == 0, e0, jnp.where(iota == 1, e1, zv))
            wx[e, pl.ds(0, 16)] = dv
            for j in range(8):
                wx[e, pl.ds(8 + 16 * j, 16)] = xls[j] * (e0 if j < 4 else e1)

    def _scat(wx, dwx, semw):
        pltpu.async_copy(wx, acc.at[dwx], semw, add=True)

    def _wait_s(wx, dwx, semw):
        pltpu.make_async_copy(wx, acc.at[dwx], semw).wait()

    pltpu.sync_copy(cidx_hbm.at[s, 0], ciA)
    _adj(ciA, caA)
    pltpu.sync_copy(cidx_hbm.at[s, 1], ciB)
    _adj(ciB, caB)
    _start_g(caA, gbA, semA)

    nstep = _NBLK // 2

    def _step(g, carry):
        pA = 2 * g
        pB = 2 * g + 1
        _start_g(caB, gbB, semB)
        _wait_g(caA, gbA, semA)

        @pl.when(g > 0)
        def _wsa():
            _wait_s(wA, dwA, semWA)
        _cp_dw(ciA, dwA)

        @pl.when(g < nstep - 1)
        def _ila():
            _idx_load(pA + 2, ciA, semIA)
        _compute(gbA, wA)
        _scat(wA, dwA, semWA)

        @pl.when(g < nstep - 1)
        def _iwa():
            _idx_wait(pA + 2, ciA, semIA)
            _adj(ciA, caA)
            _start_g(caA, gbA, semA)

        _wait_g(caB, gbB, semB)

        @pl.when(g > 0)
        def _wsb():
            _wait_s(wB, dwB, semWB)
        _cp_dw(ciB, dwB)

        @pl.when(g < nstep - 1)
        def _ilb():
            _idx_load(pB + 2, ciB, semIB)
        _compute(gbB, wB)
        _scat(wB, dwB, semWB)

        @pl.when(g < nstep - 1)
        def _iwb():
            _idx_wait(pB + 2, ciB, semIB)
            _adj(ciB, caB)
        return carry
    lax.fori_loop(0, nstep, _step, 0)
    _wait_s(wA, dwA, semWA)
    _wait_s(wB, dwB, semWB)
    plsc.subcore_barrier()

    for (o, n) in _CHUNKS:
        pltpu.sync_copy(acc.at[pl.ds(r0 + o, n)], wA.at[pl.ds(0, n)])

        @plsc.parallel_loop(0, n, unroll=2)
        def _row(r):
            denv = wA[r, pl.ds(0, 16)]
            iv = 1.0 / jnp.maximum(denv, 1e-16)
            i0 = iv[0]
            i1 = iv[1]
            for j in range(8):
                sc = i0 if j < 4 else i1
                v = wA[r, pl.ds(8 + 16 * j, 16)] * sc + biasvs[j]
                gbA[r, pl.ds(16 * j, 16)] = jnp.maximum(v, 0.0)
        pltpu.sync_copy(gbA.at[pl.ds(0, n)],
                        h1_hbm.at[pl.ds(coff + r0 + o, n)])


@functools.partial(
    pl.kernel,
    out_type=jax.ShapeDtypeStruct((_NCORE * _NP, 16), jnp.float32),
    mesh=_sc_mesh,
    compiler_params=pltpu.CompilerParams(
        needs_layout_passes=False, use_tc_tiling_on_sc=False),
    scratch_types=[
        pltpu.VMEM_SHARED((_NP, 16), jnp.float32),
        pltpu.VMEM((2 * _BLK, 32), jnp.float32),
        pltpu.VMEM((2 * _BLK, 32), jnp.float32),
        pltpu.VMEM((_BLK, 16), jnp.float32),
        pltpu.VMEM((_BLK, 16), jnp.float32),
        pltpu.VMEM((2 * _BLK,), jnp.int32),
        pltpu.VMEM((_BLK,), jnp.int32),
        pltpu.VMEM((2 * _BLK,), jnp.int32),
        pltpu.VMEM((_BLK,), jnp.int32),
        pltpu.VMEM((16,), jnp.float32),
        pltpu.SemaphoreType.DMA,
        pltpu.SemaphoreType.DMA,
        pltpu.SemaphoreType.DMA,
        pltpu.SemaphoreType.DMA,
        pltpu.SemaphoreType.DMA,
        pltpu.SemaphoreType.DMA,
    ],
)
def _l2(xlr2_hbm, cidx_hbm, att_hbm, out_hbm,
        acc2, gbA, gbB, w2A, w2B,
        ciA, dwA, ciB, dwB, attb,
        semA, semB, semWA, semWB, semIA, semIB):
    c = lax.axis_index("c")
    s = lax.axis_index("s")
    iota = lax.iota(jnp.int32, 16)
    zv = jnp.zeros((16,), jnp.float32)
    r0 = s * _RPS
    wid = s * _NCORE + c

    pltpu.sync_copy(att_hbm, attb)
    attv = attb[pl.ds(0, 16)]

    @plsc.parallel_loop(0, _BLK, unroll=4)
    def _zw(r):
        w2A[r, pl.ds(0, 16)] = zv
    for (o, n) in _CHUNKS:
        pltpu.sync_copy(w2A.at[pl.ds(0, n)], acc2.at[pl.ds(r0 + o, n)])
    plsc.subcore_barrier()

    def _idx_load(b, cix, semi):
        pltpu.async_copy(cidx_hbm.at[wid, b], cix, semi)

    def _idx_wait(b, cix, semi):
        pltpu.make_async_copy(cidx_hbm.at[wid, b], cix, semi).wait()

    def _cp_dw(cix, dwx):
        for j in range(_BLK // 16):
            dwx[pl.ds(16 * j, 16)] = cix[pl.ds(_BLK + 16 * j, 16)]

    def _start_g(cix, gbx, s1):
        pltpu.async_copy(xlr2_hbm.at[cix], gbx, s1)

    def _wait_g(cix, gbx, s1):
        pltpu.make_async_copy(xlr2_hbm.at[cix], gbx, s1).wait()

    def _compute(gbx, w2x):
        @plsc.parallel_loop(0, _BLK, unroll=2)
        def _edge(e):
            a = gbx[e, pl.ds(0, 16)]
            b8 = gbx[_BLK + e, pl.ds(8, 16)]
            v = a + b8
            f = jnp.where(v >= 0.0, v, 0.2 * v)
            exv = jnp.exp(f * attv)
            row = jnp.where(iota < 4, exv * a, jnp.where(iota < 8, exv, zv))
            w2x[e, pl.ds(0, 16)] = row

    def _scat(w2x, dwx, semw):
        pltpu.async_copy(w2x, acc2.at[dwx], semw, add=True)

    def _wait_s(w2x, dwx, semw):
        pltpu.make_async_copy(w2x, acc2.at[dwx], semw).wait()

    pltpu.sync_copy(cidx_hbm.at[wid, 0], ciA)
    pltpu.sync_copy(cidx_hbm.at[wid, 1], ciB)
    _start_g(ciA, gbA, semA)

    nstep = _NBLK2 // 2

    def _step(g, carry):
        pA = 2 * g
        pB = 2 * g + 1
        _start_g(ciB, gbB, semB)
        _wait_g(ciA, gbA, semA)

        @pl.when(g > 0)
        def _wsa():
            _wait_s(w2A, dwA, semWA)
        _cp_dw(ciA, dwA)

        @pl.when(g < nstep - 1)
        def _ila():
            _idx_load(pA + 2, ciA, semIA)
        _compute(gbA, w2A)
        _scat(w2A, dwA, semWA)

        @pl.when(g < nstep - 1)
        def _iwa():
            _idx_wait(pA + 2, ciA, semIA)
            _start_g(ciA, gbA, semA)

        _wait_g(ciB, gbB, semB)

        @pl.when(g > 0)
        def _wsb():
            _wait_s(w2B, dwB, semWB)
        _cp_dw(ciB, dwB)

        @pl.when(g < nstep - 1)
        def _ilb():
            _idx_load(pB + 2, ciB, semIB)
        _compute(gbB, w2B)
        _scat(w2B, dwB, semWB)

        @pl.when(g < nstep - 1)
        def _iwb():
            _idx_wait(pB + 2, ciB, semIB)
        return carry
    lax.fori_loop(0, nstep, _step, 0)
    _wait_s(w2A, dwA, semWA)
    _wait_s(w2B, dwB, semWB)
    plsc.subcore_barrier()

    for (o, n) in _CHUNKS:
        pltpu.sync_copy(acc2.at[pl.ds(r0 + o, n)], w2A.at[pl.ds(0, n)])
        pltpu.sync_copy(w2A.at[pl.ds(0, n)],
                        out_hbm.at[pl.ds(c * _NP + r0 + o, n)])


def _pool_body(p_ref, b_ref, b2_ref, o_ref):
    p = p_ref[0] + p_ref[1]
    num = p[:, 0:4]
    den = p[:, 4:8]
    vals = num / jnp.maximum(den, 1e-16)
    v = jnp.sum(vals, axis=1) * 0.25 + b2_ref[0, 0]
    bins = lax.broadcasted_iota(jnp.int32, (_B, _NP), 0)
    m = (b_ref[...] == bins).astype(jnp.float32)
    sums = jnp.sum(m * v[None, :], axis=1, keepdims=True)
    counts = jnp.sum(m, axis=1, keepdims=True)
    o_ref[...] = sums / jnp.maximum(counts, 1.0)


def _pool(parts, batch_p, b2v):
    return pl.pallas_call(
        _pool_body,
        grid=(1,),
        in_specs=[
            pl.BlockSpec((2, _NP, 16), lambda i: (0, 0, 0)),
            pl.BlockSpec((1, _NP), lambda i: (0, 0)),
            pl.BlockSpec((1, 1), lambda i: (0, 0)),
        ],
        out_specs=pl.BlockSpec((_B, 1), lambda i: (0, 0)),
        out_shape=jax.ShapeDtypeStruct((_B, 1), jnp.float32),
    )(parts, batch_p, b2v)


def kernel(x, edge_index, batch, Wl1, Wr1, att1, b1, Wl2, Wr2, att2, b2):
    ei = edge_index.astype(jnp.int32)
    loop = jnp.arange(_N, dtype=jnp.int32)
    src = jnp.concatenate([ei[0], loop])
    dst = jnp.concatenate([ei[1], loop])
    src_p = jnp.full((_EP,), _N, jnp.int32).at[:_ETOT].set(src)
    dst_p = jnp.full((_EP,), _N, jnp.int32).at[:_ETOT].set(dst)
    x_pad = jnp.zeros((_NP, _D), jnp.float32).at[:_N, :].set(x)

    cidx1 = jnp.concatenate(
        [src_p.reshape(_NSUB, _NBLK, _BLK),
         dst_p.reshape(_NSUB, _NBLK, _BLK) + 2 * _NP], axis=2)
    cidx2 = jnp.concatenate(
        [src_p.reshape(_NSUB * _NCORE, _NBLK2, _BLK),
         dst_p.reshape(_NSUB * _NCORE, _NBLK2, _BLK)], axis=2)

    wcat = jnp.concatenate([Wl1, Wr1], axis=1)
    xlxr = _mm1(x_pad, wcat)

    attc = att1.reshape(2, 128)
    b1c = b1.reshape(2, 128)
    h1cat = _l1(xlxr.reshape(4 * _NP, 128), cidx1, attc, b1c)
    h1 = h1cat.reshape(_NCORE, _NP, 128)

    w2cat = jnp.concatenate(
        [Wl2, Wl2, Wr2, Wr2, jnp.zeros((_D, 16), jnp.float32)], axis=1
    ).reshape(2, 128, 32)
    xlr2 = _mm2(h1, w2cat)

    att2p = (jnp.zeros((16,), jnp.float32)
             .at[:4].set(att2[:, 0]).at[4:8].set(att2[:, 0]))
    out2 = _l2(xlr2, cidx2, att2p)

    batch_p = jnp.full((1, _NP), _B, jnp.int32).at[0, :_N].set(
        batch.astype(jnp.int32))
    return _pool(out2.reshape(_NCORE, _NP, 16), batch_p, b2.reshape(1, 1))


# speedup vs baseline: 1.0488x; 1.0016x over previous
import functools

import jax
import jax.numpy as jnp
from jax import lax
from jax.experimental import pallas as pl
from jax.experimental.pallas import tpu as pltpu
from jax.experimental.pallas import tpu_sc as plsc

_N = 10000
_NP = 10112
_D = 256
_B = 64
_E = 160000
_ETOT = _E + _N
_NSUB = 16
_NCORE = 2
_BLK = 48
_NBLK = 224
_NBLK2 = 112
_EP = _NSUB * _NBLK * _BLK
_RPS = _NP // _NSUB
_ACCW = 136
_MM_BM = 1264

_sc_mesh = plsc.VectorSubcoreMesh(
    core_axis_name="c", subcore_axis_name="s",
    num_cores=_NCORE, num_subcores=_NSUB)

_CHUNKS = tuple((i * 48, 48) for i in range(13)) + ((624, 8),)


def _mm1_body(x_ref, w_ref, o_ref):
    y = jnp.dot(x_ref[...], w_ref[...], preferred_element_type=jnp.float32)
    o_ref[0] = y[:, 0:128]
    o_ref[1] = y[:, 128:256]
    o_ref[2] = y[:, 256:384]
    o_ref[3] = y[:, 384:512]


def _mm1(x_pad, wcat):
    return pl.pallas_call(
        _mm1_body,
        grid=(_NP // _MM_BM,),
        in_specs=[
            pl.BlockSpec((_MM_BM, _D), lambda i: (i, 0)),
            pl.BlockSpec((_D, 512), lambda i: (0, 0)),
        ],
        out_specs=pl.BlockSpec((4, _MM_BM, 128), lambda i: (0, i, 0)),
        out_shape=jax.ShapeDtypeStruct((4, _NP, 128), jnp.float32),
    )(x_pad, wcat)


def _mm2_body(h_ref, w_ref, o_ref):
    y0 = jnp.dot(h_ref[0], w_ref[0], preferred_element_type=jnp.float32)
    y1 = jnp.dot(h_ref[1], w_ref[1], preferred_element_type=jnp.float32)
    o_ref[...] = y0 + y1


def _mm2(h1, w2cat):
    return pl.pallas_call(
        _mm2_body,
        grid=(_NP // _MM_BM,),
        in_specs=[
            pl.BlockSpec((2, _MM_BM, 128), lambda i: (0, i, 0)),
            pl.BlockSpec((2, 128, 32), lambda i: (0, 0, 0)),
        ],
        out_specs=pl.BlockSpec((_MM_BM, 32), lambda i: (i, 0)),
        out_shape=jax.ShapeDtypeStruct((_NP, 32), jnp.float32),
    )(h1, w2cat)


@functools.partial(
    pl.kernel,
    out_type=jax.ShapeDtypeStruct((_NCORE * _NP, 128), jnp.float32),
    mesh=_sc_mesh,
    compiler_params=pltpu.CompilerParams(
        needs_layout_passes=False, use_tc_tiling_on_sc=False),
    scratch_types=[
        pltpu.VMEM_SHARED((_NP, _ACCW), jnp.float32),
        pltpu.VMEM((2 * _BLK, 128), jnp.float32),
        pltpu.VMEM((2 * _BLK, 128), jnp.float32),
        pltpu.VMEM((_BLK, _ACCW), jnp.float32),
        pltpu.VMEM((_BLK, _ACCW), jnp.float32),
        pltpu.VMEM((2 * _BLK,), jnp.int32),
        pltpu.VMEM((2 * _BLK,), jnp.int32),
        pltpu.VMEM((_BLK,), jnp.int32),
        pltpu.VMEM((2 * _BLK,), jnp.int32),
        pltpu.VMEM((2 * _BLK,), jnp.int32),
        pltpu.VMEM((_BLK,), jnp.int32),
        pltpu.VMEM((128,), jnp.float32),
        pltpu.VMEM((128,), jnp.float32),
        pltpu.SemaphoreType.DMA,
        pltpu.SemaphoreType.DMA,
        pltpu.SemaphoreType.DMA,
        pltpu.SemaphoreType.DMA,
        pltpu.SemaphoreType.DMA,
        pltpu.SemaphoreType.DMA,
    ],
)
def _l1(xlxr_hbm, cidx_hbm, att_hbm, b1_hbm, h1_hbm,
        acc, gbA, gbB, wA, wB,
        ciA, caA, dwA, ciB, caB, dwB,
        attb, biasb,
        semA, semB, semWA, semWB, semIA, semIB):
    c = lax.axis_index("c")
    s = lax.axis_index("s")
    iota = lax.iota(jnp.int32, 16)
    zv = jnp.zeros((16,), jnp.float32)
    coff = c * _NP
    r0 = s * _RPS

    pltpu.sync_copy(att_hbm.at[c], attb)
    pltpu.sync_copy(b1_hbm.at[c], biasb)
    attvs = [attb[pl.ds(16 * j, 16)] for j in range(8)]
    biasvs = [biasb[pl.ds(16 * j, 16)] for j in range(8)]

    @plsc.parallel_loop(0, _BLK, unroll=2)
    def _zw(r):
        for j in range(8):
            wA[r, pl.ds(16 * j, 16)] = zv
        wA[r, pl.ds(120, 16)] = zv
    for (o, n) in _CHUNKS:
        pltpu.sync_copy(wA.at[pl.ds(0, n)], acc.at[pl.ds(r0 + o, n)])
    plsc.subcore_barrier()

    def _idx_load(b, cix, semi):
        pltpu.async_copy(cidx_hbm.at[s, b], cix, semi)

    def _idx_wait(b, cix, semi):
        pltpu.make_async_copy(cidx_hbm.at[s, b], cix, semi).wait()

    def _adj(cix, cax):
        for j in range(2 * _BLK // 16):
            cax[pl.ds(16 * j, 16)] = cix[pl.ds(16 * j, 16)] + coff

    def _cp_dw(cix, dwx):
        for j in range(_BLK // 16):
            dwx[pl.ds(16 * j, 16)] = cix[pl.ds(_BLK + 16 * j, 16)] - 2 * _NP

    def _start_g(cax, gbx, s1):
        pltpu.async_copy(xlxr_hbm.at[cax], gbx, s1)

    def _wait_g(cax, gbx, s1):
        pltpu.make_async_copy(xlxr_hbm.at[cax], gbx, s1).wait()

    def _compute(gbx, wx):
        @plsc.parallel_loop(0, _BLK, unroll=2)
        def _edge(e):
            xls = []
            acc0 = zv
            acc1 = zv
            for j in range(8):
                xlv = gbx[e, pl.ds(16 * j, 16)]
                xrv = gbx[_BLK + e, pl.ds(16 * j, 16)]
                xls.append(xlv)
                v = xlv + xrv
                f = jnp.where(v >= 0.0, v, 0.2 * v)
                p = f * attvs[j]
                if j < 4:
                    acc0 = acc0 + p
                else:
                    acc1 = acc1 + p
            l0 = jnp.sum(acc0)
            l1 = jnp.sum(acc1)
            e0 = jnp.exp(jnp.full((16,), l0, jnp.float32))
            e1 = jnp.exp(jnp.full((16,), l1, jnp.float32))
            dv = jnp.where(iota == 0, e0, jnp.where(iota == 1, e1, zv))
            wx[e, pl.ds(0, 16)] = dv
            for j in range(8):
                wx[e, pl.ds(8 + 16 * j, 16)] = xls[j] * (e0 if j < 4 else e1)

    def _scat(wx, dwx, semw):
        pltpu.async_copy(wx, acc.at[dwx], semw, add=True)

    def _wait_s(wx, dwx, semw):
        pltpu.make_async_copy(wx, acc.at[dwx], semw).wait()

    pltpu.sync_copy(cidx_hbm.at[s, 0], ciA)
    _adj(ciA, caA)
    pltpu.sync_copy(cidx_hbm.at[s, 1], ciB)
    _adj(ciB, caB)
    _start_g(caA, gbA, semA)

    nstep = _NBLK // 2

    def _step(g, carry):
        pA = 2 * g
        pB = 2 * g + 1
        _start_g(caB, gbB, semB)
        _wait_g(caA, gbA, semA)

        @pl.when(g > 0)
        def _wsa():
            _wait_s(wA, dwA, semWA)
        _cp_dw(ciA, dwA)

        @pl.when(g < nstep - 1)
        def _ila():
            _idx_load(pA + 2, ciA, semIA)
        _compute(gbA, wA)
        _scat(wA, dwA, semWA)

        @pl.when(g < nstep - 1)
        def _iwa():
            _idx_wait(pA + 2, ciA, semIA)
            _adj(ciA, caA)
            _start_g(caA, gbA, semA)

        _wait_g(caB, gbB, semB)

        @pl.when(g > 0)
        def _wsb():
            _wait_s(wB, dwB, semWB)
        _cp_dw(ciB, dwB)

        @pl.when(g < nstep - 1)
        def _ilb():
            _idx_load(pB + 2, ciB, semIB)
        _compute(gbB, wB)
        _scat(wB, dwB, semWB)

        @pl.when(g < nstep - 1)
        def _iwb():
            _idx_wait(pB + 2, ciB, semIB)
            _adj(ciB, caB)
        return carry
    lax.fori_loop(0, nstep, _step, 0)
    _wait_s(wA, dwA, semWA)
    _wait_s(wB, dwB, semWB)
    plsc.subcore_barrier()

    for (o, n) in _CHUNKS:
        pltpu.sync_copy(acc.at[pl.ds(r0 + o, n)], wA.at[pl.ds(0, n)])

        @plsc.parallel_loop(0, n, unroll=2)
        def _row(r):
            denv = wA[r, pl.ds(0, 16)]
            iv = 1.0 / jnp.maximum(denv, 1e-16)
            i0 = iv[0]
            i1 = iv[1]
            for j in range(8):
                sc = i0 if j < 4 else i1
                v = wA[r, pl.ds(8 + 16 * j, 16)] * sc + biasvs[j]
                gbA[r, pl.ds(16 * j, 16)] = jnp.maximum(v, 0.0)
        pltpu.sync_copy(gbA.at[pl.ds(0, n)],
                        h1_hbm.at[pl.ds(coff + r0 + o, n)])


@functools.partial(
    pl.kernel,
    out_type=jax.ShapeDtypeStruct((_NCORE * _NP, 16), jnp.float32),
    mesh=_sc_mesh,
    compiler_params=pltpu.CompilerParams(
        needs_layout_passes=False, use_tc_tiling_on_sc=False),
    scratch_types=[
        pltpu.VMEM_SHARED((_NP, 16), jnp.float32),
        pltpu.VMEM((2 * _BLK, 32), jnp.float32),
        pltpu.VMEM((2 * _BLK, 32), jnp.float32),
        pltpu.VMEM((_BLK, 16), jnp.float32),
        pltpu.VMEM((_BLK, 16), jnp.float32),
        pltpu.VMEM((2 * _BLK,), jnp.int32),
        pltpu.VMEM((_BLK,), jnp.int32),
        pltpu.VMEM((2 * _BLK,), jnp.int32),
        pltpu.VMEM((_BLK,), jnp.int32),
        pltpu.VMEM((16,), jnp.float32),
        pltpu.SemaphoreType.DMA,
        pltpu.SemaphoreType.DMA,
        pltpu.SemaphoreType.DMA,
        pltpu.SemaphoreType.DMA,
        pltpu.SemaphoreType.DMA,
        pltpu.SemaphoreType.DMA,
    ],
)
def _l2(xlr2_hbm, cidx_hbm, att_hbm, out_hbm,
        acc2, gbA, gbB, w2A, w2B,
        ciA, dwA, ciB, dwB, attb,
        semA, semB, semWA, semWB, semIA, semIB):
    c = lax.axis_index("c")
    s = lax.axis_index("s")
    iota = lax.iota(jnp.int32, 16)
    zv = jnp.zeros((16,), jnp.float32)
    r0 = s * _RPS
    wid = s * _NCORE + c

    pltpu.sync_copy(att_hbm, attb)
    attv = attb[pl.ds(0, 16)]

    @plsc.parallel_loop(0, _BLK, unroll=4)
    def _zw(r):
        w2A[r, pl.ds(0, 16)] = zv
    for (o, n) in _CHUNKS:
        pltpu.sync_copy(w2A.at[pl.ds(0, n)], acc2.at[pl.ds(r0 + o, n)])
    plsc.subcore_barrier()

    def _idx_load(b, cix, semi):
        pltpu.async_copy(cidx_hbm.at[wid, b], cix, semi)

    def _idx_wait(b, cix, semi):
        pltpu.make_async_copy(cidx_hbm.at[wid, b], cix, semi).wait()

    def _cp_dw(cix, dwx):
        for j in range(_BLK // 16):
            dwx[pl.ds(16 * j, 16)] = cix[pl.ds(_BLK + 16 * j, 16)]

    def _start_g(cix, gbx, s1):
        pltpu.async_copy(xlr2_hbm.at[cix], gbx, s1)

    def _wait_g(cix, gbx, s1):
        pltpu.make_async_copy(xlr2_hbm.at[cix], gbx, s1).wait()

    def _compute(gbx, w2x):
        @plsc.parallel_loop(0, _BLK, unroll=4)
        def _edge(e):
            a = gbx[e, pl.ds(0, 16)]
            b8 = gbx[_BLK + e, pl.ds(8, 16)]
            v = a + b8
            f = jnp.where(v >= 0.0, v, 0.2 * v)
            exv = jnp.exp(f * attv)
            row = jnp.where(iota < 4, exv * a, jnp.where(iota < 8, exv, zv))
            w2x[e, pl.ds(0, 16)] = row

    def _scat(w2x, dwx, semw):
        pltpu.async_copy(w2x, acc2.at[dwx], semw, add=True)

    def _wait_s(w2x, dwx, semw):
        pltpu.make_async_copy(w2x, acc2.at[dwx], semw).wait()

    pltpu.sync_copy(cidx_hbm.at[wid, 0], ciA)
    pltpu.sync_copy(cidx_hbm.at[wid, 1], ciB)
    _start_g(ciA, gbA, semA)

    nstep = _NBLK2 // 2

    def _step(g, carry):
        pA = 2 * g
        pB = 2 * g + 1
        _start_g(ciB, gbB, semB)
        _wait_g(ciA, gbA, semA)

        @pl.when(g > 0)
        def _wsa():
            _wait_s(w2A, dwA, semWA)
        _cp_dw(ciA, dwA)

        @pl.when(g < nstep - 1)
        def _ila():
            _idx_load(pA + 2, ciA, semIA)
        _compute(gbA, w2A)
        _scat(w2A, dwA, semWA)

        @pl.when(g < nstep - 1)
        def _iwa():
            _idx_wait(pA + 2, ciA, semIA)
            _start_g(ciA, gbA, semA)

        _wait_g(ciB, gbB, semB)

        @pl.when(g > 0)
        def _wsb():
            _wait_s(w2B, dwB, semWB)
        _cp_dw(ciB, dwB)

        @pl.when(g < nstep - 1)
        def _ilb():
            _idx_load(pB + 2, ciB, semIB)
        _compute(gbB, w2B)
        _scat(w2B, dwB, semWB)

        @pl.when(g < nstep - 1)
        def _iwb():
            _idx_wait(pB + 2, ciB, semIB)
        return carry
    lax.fori_loop(0, nstep, _step, 0)
    _wait_s(w2A, dwA, semWA)
    _wait_s(w2B, dwB, semWB)
    plsc.subcore_barrier()

    for (o, n) in _CHUNKS:
        pltpu.sync_copy(acc2.at[pl.ds(r0 + o, n)], w2A.at[pl.ds(0, n)])
        pltpu.sync_copy(w2A.at[pl.ds(0, n)],
                        out_hbm.at[pl.ds(c * _NP + r0 + o, n)])


def _pool_body(p_ref, b_ref, b2_ref, o_ref):
    p = p_ref[0] + p_ref[1]
    num = p[:, 0:4]
    den = p[:, 4:8]
    vals = num / jnp.maximum(den, 1e-16)
    v = jnp.sum(vals, axis=1) * 0.25 + b2_ref[0, 0]
    bins = lax.broadcasted_iota(jnp.int32, (_B, _NP), 0)
    m = (b_ref[...] == bins).astype(jnp.float32)
    sums = jnp.sum(m * v[None, :], axis=1, keepdims=True)
    counts = jnp.sum(m, axis=1, keepdims=True)
    o_ref[...] = sums / jnp.maximum(counts, 1.0)


def _pool(parts, batch_p, b2v):
    return pl.pallas_call(
        _pool_body,
        grid=(1,),
        in_specs=[
            pl.BlockSpec((2, _NP, 16), lambda i: (0, 0, 0)),
            pl.BlockSpec((1, _NP), lambda i: (0, 0)),
            pl.BlockSpec((1, 1), lambda i: (0, 0)),
        ],
        out_specs=pl.BlockSpec((_B, 1), lambda i: (0, 0)),
        out_shape=jax.ShapeDtypeStruct((_B, 1), jnp.float32),
    )(parts, batch_p, b2v)


def kernel(x, edge_index, batch, Wl1, Wr1, att1, b1, Wl2, Wr2, att2, b2):
    ei = edge_index.astype(jnp.int32)
    loop = jnp.arange(_N, dtype=jnp.int32)
    src = jnp.concatenate([ei[0], loop])
    dst = jnp.concatenate([ei[1], loop])
    src_p = jnp.full((_EP,), _N, jnp.int32).at[:_ETOT].set(src)
    dst_p = jnp.full((_EP,), _N, jnp.int32).at[:_ETOT].set(dst)
    x_pad = jnp.zeros((_NP, _D), jnp.float32).at[:_N, :].set(x)

    cidx1 = jnp.concatenate(
        [src_p.reshape(_NSUB, _NBLK, _BLK),
         dst_p.reshape(_NSUB, _NBLK, _BLK) + 2 * _NP], axis=2)
    cidx2 = jnp.concatenate(
        [src_p.reshape(_NSUB * _NCORE, _NBLK2, _BLK),
         dst_p.reshape(_NSUB * _NCORE, _NBLK2, _BLK)], axis=2)

    wcat = jnp.concatenate([Wl1, Wr1], axis=1)
    xlxr = _mm1(x_pad, wcat)

    attc = att1.reshape(2, 128)
    b1c = b1.reshape(2, 128)
    h1cat = _l1(xlxr.reshape(4 * _NP, 128), cidx1, attc, b1c)
    h1 = h1cat.reshape(_NCORE, _NP, 128)

    w2cat = jnp.concatenate(
        [Wl2, Wl2, Wr2, Wr2, jnp.zeros((_D, 16), jnp.float32)], axis=1
    ).reshape(2, 128, 32)
    xlr2 = _mm2(h1, w2cat)

    att2p = (jnp.zeros((16,), jnp.float32)
             .at[:4].set(att2[:, 0]).at[4:8].set(att2[:, 0]))
    out2 = _l2(xlr2, cidx2, att2p)

    batch_p = jnp.full((1, _NP), _B, jnp.int32).at[0, :_N].set(
        batch.astype(jnp.int32))
    return _pool(out2.reshape(_NCORE, _NP, 16), batch_p, b2.reshape(1, 1))


# speedup vs baseline: 1.0623x; 1.0129x over previous
import functools

import jax
import jax.numpy as jnp
from jax import lax
from jax.experimental import pallas as pl
from jax.experimental.pallas import tpu as pltpu
from jax.experimental.pallas import tpu_sc as plsc

_N = 10000
_NP = 10112
_D = 256
_B = 64
_E = 160000
_ETOT = _E + _N
_NSUB = 16
_NCORE = 2
_BLK = 48
_NBLK = 224
_BLK2 = 64
_NBLK2 = 84
_EP = _NSUB * _NBLK * _BLK
_RPS = _NP // _NSUB
_ACCW = 136
_MM_BM = 1264

_sc_mesh = plsc.VectorSubcoreMesh(
    core_axis_name="c", subcore_axis_name="s",
    num_cores=_NCORE, num_subcores=_NSUB)

_CHUNKS = tuple((i * 48, 48) for i in range(13)) + ((624, 8),)


def _mm1_body(x_ref, w_ref, o_ref):
    y = jnp.dot(x_ref[...], w_ref[...], preferred_element_type=jnp.float32)
    o_ref[0] = y[:, 0:128]
    o_ref[1] = y[:, 128:256]
    o_ref[2] = y[:, 256:384]
    o_ref[3] = y[:, 384:512]


def _mm1(x_pad, wcat):
    return pl.pallas_call(
        _mm1_body,
        grid=(_NP // _MM_BM,),
        in_specs=[
            pl.BlockSpec((_MM_BM, _D), lambda i: (i, 0)),
            pl.BlockSpec((_D, 512), lambda i: (0, 0)),
        ],
        out_specs=pl.BlockSpec((4, _MM_BM, 128), lambda i: (0, i, 0)),
        out_shape=jax.ShapeDtypeStruct((4, _NP, 128), jnp.float32),
    )(x_pad, wcat)


def _mm2_body(h_ref, w_ref, o_ref):
    y0 = jnp.dot(h_ref[0], w_ref[0], preferred_element_type=jnp.float32)
    y1 = jnp.dot(h_ref[1], w_ref[1], preferred_element_type=jnp.float32)
    o_ref[...] = y0 + y1


def _mm2(h1, w2cat):
    return pl.pallas_call(
        _mm2_body,
        grid=(_NP // _MM_BM,),
        in_specs=[
            pl.BlockSpec((2, _MM_BM, 128), lambda i: (0, i, 0)),
            pl.BlockSpec((2, 128, 32), lambda i: (0, 0, 0)),
        ],
        out_specs=pl.BlockSpec((_MM_BM, 32), lambda i: (i, 0)),
        out_shape=jax.ShapeDtypeStruct((_NP, 32), jnp.float32),
    )(h1, w2cat)


@functools.partial(
    pl.kernel,
    out_type=jax.ShapeDtypeStruct((_NCORE * _NP, 128), jnp.float32),
    mesh=_sc_mesh,
    compiler_params=pltpu.CompilerParams(
        needs_layout_passes=False, use_tc_tiling_on_sc=False),
    scratch_types=[
        pltpu.VMEM_SHARED((_NP, _ACCW), jnp.float32),
        pltpu.VMEM((2 * _BLK, 128), jnp.float32),
        pltpu.VMEM((2 * _BLK, 128), jnp.float32),
        pltpu.VMEM((_BLK, _ACCW), jnp.float32),
        pltpu.VMEM((_BLK, _ACCW), jnp.float32),
        pltpu.VMEM((2 * _BLK,), jnp.int32),
        pltpu.VMEM((2 * _BLK,), jnp.int32),
        pltpu.VMEM((_BLK,), jnp.int32),
        pltpu.VMEM((2 * _BLK,), jnp.int32),
        pltpu.VMEM((2 * _BLK,), jnp.int32),
        pltpu.VMEM((_BLK,), jnp.int32),
        pltpu.VMEM((128,), jnp.float32),
        pltpu.VMEM((128,), jnp.float32),
        pltpu.SemaphoreType.DMA,
        pltpu.SemaphoreType.DMA,
        pltpu.SemaphoreType.DMA,
        pltpu.SemaphoreType.DMA,
        pltpu.SemaphoreType.DMA,
        pltpu.SemaphoreType.DMA,
    ],
)
def _l1(xlxr_hbm, cidx_hbm, att_hbm, b1_hbm, h1_hbm,
        acc, gbA, gbB, wA, wB,
        ciA, caA, dwA, ciB, caB, dwB,
        attb, biasb,
        semA, semB, semWA, semWB, semIA, semIB):
    c = lax.axis_index("c")
    s = lax.axis_index("s")
    iota = lax.iota(jnp.int32, 16)
    zv = jnp.zeros((16,), jnp.float32)
    coff = c * _NP
    r0 = s * _RPS

    pltpu.sync_copy(att_hbm.at[c], attb)
    pltpu.sync_copy(b1_hbm.at[c], biasb)
    attvs = [attb[pl.ds(16 * j, 16)] for j in range(8)]
    biasvs = [biasb[pl.ds(16 * j, 16)] for j in range(8)]

    @plsc.parallel_loop(0, _BLK, unroll=2)
    def _zw(r):
        for j in range(8):
            wA[r, pl.ds(16 * j, 16)] = zv
        wA[r, pl.ds(120, 16)] = zv
    for (o, n) in _CHUNKS:
        pltpu.sync_copy(wA.at[pl.ds(0, n)], acc.at[pl.ds(r0 + o, n)])
    plsc.subcore_barrier()

    def _idx_load(b, cix, semi):
        pltpu.async_copy(cidx_hbm.at[s, b], cix, semi)

    def _idx_wait(b, cix, semi):
        pltpu.make_async_copy(cidx_hbm.at[s, b], cix, semi).wait()

    def _adj(cix, cax):
        for j in range(2 * _BLK // 16):
            cax[pl.ds(16 * j, 16)] = cix[pl.ds(16 * j, 16)] + coff

    def _cp_dw(cix, dwx):
        for j in range(_BLK // 16):
            dwx[pl.ds(16 * j, 16)] = cix[pl.ds(_BLK + 16 * j, 16)] - 2 * _NP

    def _start_g(cax, gbx, s1):
        pltpu.async_copy(xlxr_hbm.at[cax], gbx, s1)

    def _wait_g(cax, gbx, s1):
        pltpu.make_async_copy(xlxr_hbm.at[cax], gbx, s1).wait()

    def _compute(gbx, wx):
        @plsc.parallel_loop(0, _BLK, unroll=2)
        def _edge(e):
            xls = []
            acc0 = zv
            acc1 = zv
            for j in range(8):
                xlv = gbx[e, pl.ds(16 * j, 16)]
                xrv = gbx[_BLK + e, pl.ds(16 * j, 16)]
                xls.append(xlv)
                v = xlv + xrv
                f = jnp.where(v >= 0.0, v, 0.2 * v)
                p = f * attvs[j]
                if j < 4:
                    acc0 = acc0 + p
                else:
                    acc1 = acc1 + p
            l0 = jnp.sum(acc0)
            l1 = jnp.sum(acc1)
            e0 = jnp.exp(jnp.full((16,), l0, jnp.float32))
            e1 = jnp.exp(jnp.full((16,), l1, jnp.float32))
            dv = jnp.where(iota == 0, e0, jnp.where(iota == 1, e1, zv))
            wx[e, pl.ds(0, 16)] = dv
            for j in range(8):
                wx[e, pl.ds(8 + 16 * j, 16)] = xls[j] * (e0 if j < 4 else e1)

    def _scat(wx, dwx, semw):
        pltpu.async_copy(wx, acc.at[dwx], semw, add=True)

    def _wait_s(wx, dwx, semw):
        pltpu.make_async_copy(wx, acc.at[dwx], semw).wait()

    pltpu.sync_copy(cidx_hbm.at[s, 0], ciA)
    _adj(ciA, caA)
    pltpu.sync_copy(cidx_hbm.at[s, 1], ciB)
    _adj(ciB, caB)
    _start_g(caA, gbA, semA)

    nstep = _NBLK // 2

    def _step(g, carry):
        pA = 2 * g
        pB = 2 * g + 1
        _start_g(caB, gbB, semB)
        _wait_g(caA, gbA, semA)

        @pl.when(g > 0)
        def _wsa():
            _wait_s(wA, dwA, semWA)
        _cp_dw(ciA, dwA)

        @pl.when(g < nstep - 1)
        def _ila():
            _idx_load(pA + 2, ciA, semIA)
        _compute(gbA, wA)
        _scat(wA, dwA, semWA)

        @pl.when(g < nstep - 1)
        def _iwa():
            _idx_wait(pA + 2, ciA, semIA)
            _adj(ciA, caA)
            _start_g(caA, gbA, semA)

        _wait_g(caB, gbB, semB)

        @pl.when(g > 0)
        def _wsb():
            _wait_s(wB, dwB, semWB)
        _cp_dw(ciB, dwB)

        @pl.when(g < nstep - 1)
        def _ilb():
            _idx_load(pB + 2, ciB, semIB)
        _compute(gbB, wB)
        _scat(wB, dwB, semWB)

        @pl.when(g < nstep - 1)
        def _iwb():
            _idx_wait(pB + 2, ciB, semIB)
            _adj(ciB, caB)
        return carry
    lax.fori_loop(0, nstep, _step, 0)
    _wait_s(wA, dwA, semWA)
    _wait_s(wB, dwB, semWB)
    plsc.subcore_barrier()

    for (o, n) in _CHUNKS:
        pltpu.sync_copy(acc.at[pl.ds(r0 + o, n)], wA.at[pl.ds(0, n)])

        @plsc.parallel_loop(0, n, unroll=2)
        def _row(r):
            denv = wA[r, pl.ds(0, 16)]
            iv = 1.0 / jnp.maximum(denv, 1e-16)
            i0 = iv[0]
            i1 = iv[1]
            for j in range(8):
                sc = i0 if j < 4 else i1
                v = wA[r, pl.ds(8 + 16 * j, 16)] * sc + biasvs[j]
                gbA[r, pl.ds(16 * j, 16)] = jnp.maximum(v, 0.0)
        pltpu.sync_copy(gbA.at[pl.ds(0, n)],
                        h1_hbm.at[pl.ds(coff + r0 + o, n)])


@functools.partial(
    pl.kernel,
    out_type=jax.ShapeDtypeStruct((_NCORE * _NP, 16), jnp.float32),
    mesh=_sc_mesh,
    compiler_params=pltpu.CompilerParams(
        needs_layout_passes=False, use_tc_tiling_on_sc=False),
    scratch_types=[
        pltpu.VMEM_SHARED((_NP, 16), jnp.float32),
        pltpu.VMEM((2 * _BLK2, 32), jnp.float32),
        pltpu.VMEM((2 * _BLK2, 32), jnp.float32),
        pltpu.VMEM((_BLK2, 16), jnp.float32),
        pltpu.VMEM((_BLK2, 16), jnp.float32),
        pltpu.VMEM((2 * _BLK2,), jnp.int32),
        pltpu.VMEM((_BLK2,), jnp.int32),
        pltpu.VMEM((2 * _BLK2,), jnp.int32),
        pltpu.VMEM((_BLK2,), jnp.int32),
        pltpu.VMEM((16,), jnp.float32),
        pltpu.SemaphoreType.DMA,
        pltpu.SemaphoreType.DMA,
        pltpu.SemaphoreType.DMA,
        pltpu.SemaphoreType.DMA,
        pltpu.SemaphoreType.DMA,
        pltpu.SemaphoreType.DMA,
    ],
)
def _l2(xlr2_hbm, cidx_hbm, att_hbm, out_hbm,
        acc2, gbA, gbB, w2A, w2B,
        ciA, dwA, ciB, dwB, attb,
        semA, semB, semWA, semWB, semIA, semIB):
    c = lax.axis_index("c")
    s = lax.axis_index("s")
    iota = lax.iota(jnp.int32, 16)
    zv = jnp.zeros((16,), jnp.float32)
    r0 = s * _RPS
    wid = s * _NCORE + c

    pltpu.sync_copy(att_hbm, attb)
    attv = attb[pl.ds(0, 16)]

    @plsc.parallel_loop(0, _BLK2, unroll=4)
    def _zw(r):
        w2A[r, pl.ds(0, 16)] = zv
    for (o, n) in _CHUNKS:
        pltpu.sync_copy(w2A.at[pl.ds(0, n)], acc2.at[pl.ds(r0 + o, n)])
    plsc.subcore_barrier()

    def _idx_load(b, cix, semi):
        pltpu.async_copy(cidx_hbm.at[wid, b], cix, semi)

    def _idx_wait(b, cix, semi):
        pltpu.make_async_copy(cidx_hbm.at[wid, b], cix, semi).wait()

    def _cp_dw(cix, dwx):
        for j in range(_BLK2 // 16):
            dwx[pl.ds(16 * j, 16)] = cix[pl.ds(_BLK2 + 16 * j, 16)]

    def _start_g(cix, gbx, s1):
        pltpu.async_copy(xlr2_hbm.at[cix], gbx, s1)

    def _wait_g(cix, gbx, s1):
        pltpu.make_async_copy(xlr2_hbm.at[cix], gbx, s1).wait()

    def _compute(gbx, w2x):
        @plsc.parallel_loop(0, _BLK2, unroll=4)
        def _edge(e):
            a = gbx[e, pl.ds(0, 16)]
            b8 = gbx[_BLK2 + e, pl.ds(8, 16)]
            v = a + b8
            f = jnp.where(v >= 0.0, v, 0.2 * v)
            exv = jnp.exp(f * attv)
            row = jnp.where(iota < 4, exv * a, jnp.where(iota < 8, exv, zv))
            w2x[e, pl.ds(0, 16)] = row

    def _scat(w2x, dwx, semw):
        pltpu.async_copy(w2x, acc2.at[dwx], semw, add=True)

    def _wait_s(w2x, dwx, semw):
        pltpu.make_async_copy(w2x, acc2.at[dwx], semw).wait()

    pltpu.sync_copy(cidx_hbm.at[wid, 0], ciA)
    pltpu.sync_copy(cidx_hbm.at[wid, 1], ciB)
    _start_g(ciA, gbA, semA)

    nstep = _NBLK2 // 2

    def _step(g, carry):
        pA = 2 * g
        pB = 2 * g + 1
        _start_g(ciB, gbB, semB)
        _wait_g(ciA, gbA, semA)

        @pl.when(g > 0)
        def _wsa():
            _wait_s(w2A, dwA, semWA)
        _cp_dw(ciA, dwA)

        @pl.when(g < nstep - 1)
        def _ila():
            _idx_load(pA + 2, ciA, semIA)
        _compute(gbA, w2A)
        _scat(w2A, dwA, semWA)

        @pl.when(g < nstep - 1)
        def _iwa():
            _idx_wait(pA + 2, ciA, semIA)
            _start_g(ciA, gbA, semA)

        _wait_g(ciB, gbB, semB)

        @pl.when(g > 0)
        def _wsb():
            _wait_s(w2B, dwB, semWB)
        _cp_dw(ciB, dwB)

        @pl.when(g < nstep - 1)
        def _ilb():
            _idx_load(pB + 2, ciB, semIB)
        _compute(gbB, w2B)
        _scat(w2B, dwB, semWB)

        @pl.when(g < nstep - 1)
        def _iwb():
            _idx_wait(pB + 2, ciB, semIB)
        return carry
    lax.fori_loop(0, nstep, _step, 0)
    _wait_s(w2A, dwA, semWA)
    _wait_s(w2B, dwB, semWB)
    plsc.subcore_barrier()

    for (o, n) in _CHUNKS:
        pltpu.sync_copy(acc2.at[pl.ds(r0 + o, n)], w2A.at[pl.ds(0, n)])
        pltpu.sync_copy(w2A.at[pl.ds(0, n)],
                        out_hbm.at[pl.ds(c * _NP + r0 + o, n)])


def _pool_body(p_ref, b_ref, b2_ref, o_ref):
    p = p_ref[0] + p_ref[1]
    num = p[:, 0:4]
    den = p[:, 4:8]
    vals = num / jnp.maximum(den, 1e-16)
    v = jnp.sum(vals, axis=1) * 0.25 + b2_ref[0, 0]
    bins = lax.broadcasted_iota(jnp.int32, (_B, _NP), 0)
    m = (b_ref[...] == bins).astype(jnp.float32)
    sums = jnp.sum(m * v[None, :], axis=1, keepdims=True)
    counts = jnp.sum(m, axis=1, keepdims=True)
    o_ref[...] = sums / jnp.maximum(counts, 1.0)


def _pool(parts, batch_p, b2v):
    return pl.pallas_call(
        _pool_body,
        grid=(1,),
        in_specs=[
            pl.BlockSpec((2, _NP, 16), lambda i: (0, 0, 0)),
            pl.BlockSpec((1, _NP), lambda i: (0, 0)),
            pl.BlockSpec((1, 1), lambda i: (0, 0)),
        ],
        out_specs=pl.BlockSpec((_B, 1), lambda i: (0, 0)),
        out_shape=jax.ShapeDtypeStruct((_B, 1), jnp.float32),
    )(parts, batch_p, b2v)


def kernel(x, edge_index, batch, Wl1, Wr1, att1, b1, Wl2, Wr2, att2, b2):
    ei = edge_index.astype(jnp.int32)
    loop = jnp.arange(_N, dtype=jnp.int32)
    src = jnp.concatenate([ei[0], loop])
    dst = jnp.concatenate([ei[1], loop])
    src_p = jnp.full((_EP,), _N, jnp.int32).at[:_ETOT].set(src)
    dst_p = jnp.full((_EP,), _N, jnp.int32).at[:_ETOT].set(dst)
    x_pad = jnp.zeros((_NP, _D), jnp.float32).at[:_N, :].set(x)

    cidx1 = jnp.concatenate(
        [src_p.reshape(_NSUB, _NBLK, _BLK),
         dst_p.reshape(_NSUB, _NBLK, _BLK) + 2 * _NP], axis=2)
    cidx2 = jnp.concatenate(
        [src_p.reshape(_NSUB * _NCORE, _NBLK2, _BLK2),
         dst_p.reshape(_NSUB * _NCORE, _NBLK2, _BLK2)], axis=2)

    wcat = jnp.concatenate([Wl1, Wr1], axis=1)
    xlxr = _mm1(x_pad, wcat)

    attc = att1.reshape(2, 128)
    b1c = b1.reshape(2, 128)
    h1cat = _l1(xlxr.reshape(4 * _NP, 128), cidx1, attc, b1c)
    h1 = h1cat.reshape(_NCORE, _NP, 128)

    w2cat = jnp.concatenate(
        [Wl2, Wl2, Wr2, Wr2, jnp.zeros((_D, 16), jnp.float32)], axis=1
    ).reshape(2, 128, 32)
    xlr2 = _mm2(h1, w2cat)

    att2p = (jnp.zeros((16,), jnp.float32)
             .at[:4].set(att2[:, 0]).at[4:8].set(att2[:, 0]))
    out2 = _l2(xlr2, cidx2, att2p)

    batch_p = jnp.full((1, _NP), _B, jnp.int32).at[0, :_N].set(
        batch.astype(jnp.int32))
    return _pool(out2.reshape(_NCORE, _NP, 16), batch_p, b2.reshape(1, 1))
